# baseline (speedup 1.0000x reference)
import jax
import jax.numpy as jnp
from jax import lax
from jax.experimental import pallas as pl
from jax.experimental.pallas import tpu as pltpu
from jax.experimental.pallas import tpu_sc as plsc

DIM = 32
N_FIELDS = 26
BATCH = 16384
NUM_CORES = 2
NUM_SUBCORES = 16
NUM_WORKERS = NUM_CORES * NUM_SUBCORES
BLK = 128
N_BLOCKS = N_FIELDS * (BATCH // BLK)
BLOCKS_PER_W = N_BLOCKS // NUM_WORKERS
IDX_PER_W = BLOCKS_PER_W * BLK
P_ROWS = N_FIELDS * (DIM // 8) * (BATCH // BLK) * 8
TC_PER_F = BATCH // BLK
ROWS_PER_F = (DIM // 8) * TC_PER_F * 8

_mesh = plsc.VectorSubcoreMesh(core_axis_name="c", subcore_axis_name="s")


def _body(table_hbm, idx_hbm, p_hbm, idx_v, rows_a, rows_b, t_a, t_b, sem_g, sem_w):
    wid = lax.axis_index("s") * NUM_CORES + lax.axis_index("c")
    base_blk = wid * BLOCKS_PER_W

    pltpu.sync_copy(idx_hbm.at[pl.ds(wid * IDX_PER_W, IDX_PER_W)], idx_v)

    lane = lax.iota(jnp.int32, 16)
    rowsel = [lane + 16 * k for k in range(8)]

    def gather_desc(z, rows_v):
        return pltpu.make_async_copy(
            table_hbm.at[idx_v.at[pl.ds(z * BLK, BLK)]], rows_v, sem_g
        )

    def write_descs(z, t_v):
        blk = base_blk + z
        f2 = blk // TC_PER_F
        tc = blk % TC_PER_F
        row0 = f2 * ROWS_PER_F + tc * 8
        return [
            pltpu.make_async_copy(
                t_v.at[pl.ds(tr * 8, 8), :],
                p_hbm.at[pl.ds(row0 + tr * TC_PER_F * 8, 8), :],
                sem_w,
            )
            for tr in range(4)
        ]

    def transpose_block(rows_v, t_v):
        @plsc.parallel_loop(0, DIM, 1, unroll=8)
        def _(d):
            col = jnp.full((16,), 0, jnp.int32) + d
            for k in range(8):
                src = plsc.load_gather(rows_v, [rowsel[k], col])
                t_v[d, pl.ds(16 * k, 16)] = src

    gather_desc(0, rows_a).start()

    def loop_body(zz, carry):
        for sub, rows_v, t_v in ((0, rows_a, t_a), (1, rows_b, t_b)):
            z = 2 * zz + sub
            gather_desc(z, rows_v).wait()

            @pl.when(z + 1 < BLOCKS_PER_W)
            def _():
                gather_desc(z + 1, rows_b if sub == 0 else rows_a).start()

            @pl.when(z >= 2)
            def _():
                for d in write_descs(z - 2, t_v):
                    d.wait()

            transpose_block(rows_v, t_v)
            for d in write_descs(z, t_v):
                d.start()
        return carry

    lax.fori_loop(0, BLOCKS_PER_W // 2, loop_body, 0)
    for d in write_descs(BLOCKS_PER_W - 2, t_a):
        d.wait()
    for d in write_descs(BLOCKS_PER_W - 1, t_b):
        d.wait()


@jax.jit
def _embed(idx_flat_f, emb):
    k = pl.kernel(
        _body,
        mesh=_mesh,
        out_type=jax.ShapeDtypeStruct((P_ROWS, BLK), jnp.float32),
        scratch_types=[
            pltpu.VMEM((IDX_PER_W,), jnp.int32),
            pltpu.VMEM((BLK, DIM), jnp.float32),
            pltpu.VMEM((BLK, DIM), jnp.float32),
            pltpu.VMEM((DIM, BLK), jnp.float32),
            pltpu.VMEM((DIM, BLK), jnp.float32),
            pltpu.SemaphoreType.DMA,
            pltpu.SemaphoreType.DMA,
        ],
        compiler_params=pltpu.CompilerParams(
            use_tc_tiling_on_sc=False, needs_layout_passes=False
        ),
    )
    return k(emb, idx_flat_f)


def kernel(token_ids, emb):
    idx_flat_f = token_ids.T.reshape(-1)
    p = _embed(idx_flat_f, emb)
    p5 = p.reshape(N_FIELDS, DIM // 8, TC_PER_F, 8, BLK)
    return p5.transpose(2, 4, 0, 1, 3).reshape(BATCH, N_FIELDS, DIM)


# speedup vs baseline: 1.0196x; 1.0196x over previous
import jax
import jax.numpy as jnp
from jax import lax
from jax.experimental import pallas as pl
from jax.experimental.pallas import tpu as pltpu
from jax.experimental.pallas import tpu_sc as plsc

DIM = 32
N_FIELDS = 26
BATCH = 16384
NUM_CORES = 2
NUM_SUBCORES = 16
NUM_WORKERS = NUM_CORES * NUM_SUBCORES
BLK = 128
TC_PER_F = BATCH // BLK
TC_PER_W = TC_PER_F // NUM_WORKERS
BLOCKS_PER_W = N_FIELDS * TC_PER_W
P_ROWS = N_FIELDS * (DIM // 8) * TC_PER_F * 8
ROWS_PER_F = (DIM // 8) * TC_PER_F * 8

_mesh = plsc.VectorSubcoreMesh(core_axis_name="c", subcore_axis_name="s")


def _body(table_hbm, idx_hbm, p_hbm, idx_v, rows_a, rows_b, t_a, t_b, sem_g, sem_w):
    wid = lax.axis_index("s") * NUM_CORES + lax.axis_index("c")

    pltpu.sync_copy(idx_hbm.at[:, pl.ds(wid * (TC_PER_W * BLK), TC_PER_W * BLK)], idx_v)

    lane = lax.iota(jnp.int32, 16)
    rowsel = [lane + 16 * k for k in range(8)]

    def gather_desc(z, rows_v):
        f2 = z // TC_PER_W
        t = z % TC_PER_W
        return pltpu.make_async_copy(
            table_hbm.at[idx_v.at[f2, pl.ds(t * BLK, BLK)]], rows_v, sem_g
        )

    def write_descs(z, t_v):
        f2 = z // TC_PER_W
        tc = wid * TC_PER_W + z % TC_PER_W
        row0 = f2 * ROWS_PER_F + tc * 8
        return [
            pltpu.make_async_copy(
                t_v.at[pl.ds(tr * 8, 8), :],
                p_hbm.at[pl.ds(row0 + tr * TC_PER_F * 8, 8), :],
                sem_w,
            )
            for tr in range(4)
        ]

    def transpose_block(rows_v, t_v):
        @plsc.parallel_loop(0, DIM, 1, unroll=4)
        def _(d):
            col = jnp.full((16,), 0, jnp.int32) + d
            for k in range(8):
                src = plsc.load_gather(rows_v, [rowsel[k], col])
                t_v[d, pl.ds(16 * k, 16)] = src

    gather_desc(0, rows_a).start()

    def loop_body(zz, carry):
        for sub, rows_v, t_v in ((0, rows_a, t_a), (1, rows_b, t_b)):
            z = 2 * zz + sub
            gather_desc(z, rows_v).wait()

            @pl.when(z + 1 < BLOCKS_PER_W)
            def _():
                gather_desc(z + 1, rows_b if sub == 0 else rows_a).start()

            @pl.when(z >= 2)
            def _():
                for d in write_descs(z - 2, t_v):
                    d.wait()

            transpose_block(rows_v, t_v)
            for d in write_descs(z, t_v):
                d.start()
        return carry

    lax.fori_loop(0, BLOCKS_PER_W // 2, loop_body, 0)
    for d in write_descs(BLOCKS_PER_W - 2, t_a):
        d.wait()
    for d in write_descs(BLOCKS_PER_W - 1, t_b):
        d.wait()


@jax.jit
def _embed(idx_t, emb):
    k = pl.kernel(
        _body,
        mesh=_mesh,
        out_type=jax.ShapeDtypeStruct((P_ROWS, BLK), jnp.float32),
        scratch_types=[
            pltpu.VMEM((N_FIELDS, TC_PER_W * BLK), jnp.int32),
            pltpu.VMEM((BLK, DIM), jnp.float32),
            pltpu.VMEM((BLK, DIM), jnp.float32),
            pltpu.VMEM((DIM, BLK), jnp.float32),
            pltpu.VMEM((DIM, BLK), jnp.float32),
            pltpu.SemaphoreType.DMA,
            pltpu.SemaphoreType.DMA,
        ],
        compiler_params=pltpu.CompilerParams(
            use_tc_tiling_on_sc=False, needs_layout_passes=False
        ),
    )
    return k(emb, idx_t)


def kernel(token_ids, emb):
    p = _embed(token_ids.T, emb)
    p5 = p.reshape(N_FIELDS, DIM // 8, TC_PER_F, 8, BLK)
    return p5.transpose(2, 4, 0, 1, 3).reshape(BATCH, N_FIELDS, DIM)
